# baseline (speedup 1.0000x reference)
import functools

import jax
import jax.numpy as jnp
from jax import lax
from jax.experimental import pallas as pl
from jax.experimental.pallas import tpu as pltpu
from jax.experimental.pallas import tpu_sc as plsc

B = 16384
D = 64
_LANES = 16


def _make_sc_bias():
    info = plsc.get_sparse_core_info()
    nc, ns = info.num_cores, info.num_subcores
    nw = nc * ns
    bpw = B // nw
    mesh = plsc.VectorSubcoreMesh(core_axis_name="c", subcore_axis_name="s")

    @functools.partial(
        pl.kernel,
        mesh=mesh,
        out_type=jax.ShapeDtypeStruct((B,), jnp.float32),
        scratch_types=[
            pltpu.VMEM((bpw,), jnp.int32),
            pltpu.VMEM((bpw,), jnp.int32),
            pltpu.VMEM((bpw,), jnp.float32),
            pltpu.VMEM((bpw,), jnp.float32),
            pltpu.SemaphoreType.DMA,
            pltpu.SemaphoreType.DMA,
        ],
    )
    def sc_bias(uid_hbm, iid_hbm, bu_hbm, bi_hbm, out_hbm,
                uidx, iidx, urows, irows, sem_u, sem_i):
        wid = lax.axis_index("s") * nc + lax.axis_index("c")
        base = wid * bpw
        pltpu.sync_copy(uid_hbm.at[pl.ds(base, bpw)], uidx)
        pltpu.sync_copy(iid_hbm.at[pl.ds(base, bpw)], iidx)
        cu = pltpu.async_copy(bu_hbm.at[uidx], urows, sem_u)
        ci = pltpu.async_copy(bi_hbm.at[iidx], irows, sem_i)
        cu.wait()
        ci.wait()
        for j in range(bpw // _LANES):
            sl = pl.ds(j * _LANES, _LANES)
            urows[sl] = urows[sl] + irows[sl]
        pltpu.sync_copy(urows, out_hbm.at[pl.ds(base, bpw)])

    return sc_bias


_sc_bias = _make_sc_bias()

_TC_BLK = 2048


def _tc_body(w_ref, b_ref, f_ref, bias_ref, o_ref):
    w = w_ref[...]
    f = f_ref[...]
    o_ref[...] = jnp.sum(f * w, axis=1, keepdims=True) + b_ref[0, 0] + bias_ref[...]


def _tc_matvec(fc_weight, fc_bias2, feature, bias2):
    return pl.pallas_call(
        _tc_body,
        grid=(B // _TC_BLK,),
        in_specs=[
            pl.BlockSpec((1, D), lambda i: (0, 0)),
            pl.BlockSpec((1, 1), lambda i: (0, 0)),
            pl.BlockSpec((_TC_BLK, D), lambda i: (i, 0)),
            pl.BlockSpec((_TC_BLK, 1), lambda i: (i, 0)),
        ],
        out_specs=pl.BlockSpec((_TC_BLK, 1), lambda i: (i, 0)),
        out_shape=jax.ShapeDtypeStruct((B, 1), jnp.float32),
    )(fc_weight, fc_bias2, feature, bias2)


def kernel(feature, user_id, item_id, fc_weight, fc_bias, b_users, b_items):
    bias1 = _sc_bias(user_id, item_id,
                     b_users.reshape(-1), b_items.reshape(-1))
    return _tc_matvec(fc_weight, fc_bias.reshape(1, 1), feature,
                      bias1.reshape(B, 1))


# speedup vs baseline: 2.0082x; 2.0082x over previous
import functools

import jax
import jax.numpy as jnp
from jax import lax
from jax.experimental import pallas as pl
from jax.experimental.pallas import tpu as pltpu
from jax.experimental.pallas import tpu_sc as plsc

B = 16384
D = 64
_LANES = 16


def _make_sc_combine():
    info = plsc.get_sparse_core_info()
    nc, ns = info.num_cores, info.num_subcores
    nw = nc * ns
    bpw = B // nw
    mesh = plsc.VectorSubcoreMesh(core_axis_name="c", subcore_axis_name="s")

    @functools.partial(
        pl.kernel,
        mesh=mesh,
        out_type=jax.ShapeDtypeStruct((B,), jnp.float32),
        scratch_types=[
            pltpu.VMEM((bpw,), jnp.int32),
            pltpu.VMEM((bpw,), jnp.int32),
            pltpu.VMEM((bpw,), jnp.float32),
            pltpu.VMEM((bpw,), jnp.float32),
            pltpu.VMEM((bpw,), jnp.float32),
            pltpu.SemaphoreType.DMA,
            pltpu.SemaphoreType.DMA,
        ],
    )
    def sc_combine(uid_hbm, iid_hbm, bu_hbm, bi_hbm, fc_hbm, out_hbm,
                   uidx, iidx, urows, irows, fcv, sem_u, sem_i):
        wid = lax.axis_index("s") * nc + lax.axis_index("c")
        base = wid * bpw
        pltpu.sync_copy(uid_hbm.at[pl.ds(base, bpw)], uidx)
        pltpu.sync_copy(iid_hbm.at[pl.ds(base, bpw)], iidx)
        cu = pltpu.async_copy(bu_hbm.at[uidx], urows, sem_u)
        ci = pltpu.async_copy(bi_hbm.at[iidx], irows, sem_i)
        pltpu.sync_copy(fc_hbm.at[pl.ds(base, bpw)], fcv)
        cu.wait()
        ci.wait()
        for j in range(bpw // _LANES):
            sl = pl.ds(j * _LANES, _LANES)
            fcv[sl] = fcv[sl] + urows[sl] + irows[sl]
        pltpu.sync_copy(fcv, out_hbm.at[pl.ds(base, bpw)])

    return sc_combine


_sc_combine = _make_sc_combine()

_TC_BLK = 2048


def _tc_body(w_ref, b_ref, f_ref, o_ref):
    o_ref[...] = jnp.sum(f_ref[...] * w_ref[...], axis=1) + b_ref[0, 0]


def _tc_matvec(fc_weight, fc_bias2, feature):
    return pl.pallas_call(
        _tc_body,
        grid=(B // _TC_BLK,),
        in_specs=[
            pl.BlockSpec((1, D), lambda i: (0, 0)),
            pl.BlockSpec((1, 1), lambda i: (0, 0)),
            pl.BlockSpec((_TC_BLK, D), lambda i: (i, 0)),
        ],
        out_specs=pl.BlockSpec((_TC_BLK,), lambda i: (i,)),
        out_shape=jax.ShapeDtypeStruct((B,), jnp.float32),
    )(fc_weight, fc_bias2, feature)


def _flatten_table(t):
    n = t.shape[0]
    npad = (-n) % 1024
    return jnp.pad(t, ((0, npad), (0, 0))).reshape(-1)


def kernel(feature, user_id, item_id, fc_weight, fc_bias, b_users, b_items):
    fc1d = _tc_matvec(fc_weight, fc_bias.reshape(1, 1), feature)
    out1d = _sc_combine(user_id, item_id,
                        _flatten_table(b_users), _flatten_table(b_items), fc1d)
    return out1d.reshape(B, 1)


# speedup vs baseline: 2.4208x; 1.2055x over previous
import functools

import jax
import jax.numpy as jnp
from jax import lax
from jax.experimental import pallas as pl
from jax.experimental.pallas import tpu as pltpu
from jax.experimental.pallas import tpu_sc as plsc

B = 16384
D = 64
_LANES = 16


def _make_sc_combine():
    info = plsc.get_sparse_core_info()
    nc, ns = info.num_cores, info.num_subcores
    nw = nc * ns
    bpw = B // nw
    mesh = plsc.VectorSubcoreMesh(core_axis_name="c", subcore_axis_name="s")

    @functools.partial(
        pl.kernel,
        mesh=mesh,
        out_type=jax.ShapeDtypeStruct((B,), jnp.float32),
        scratch_types=[
            pltpu.VMEM((bpw,), jnp.int32),
            pltpu.VMEM((bpw,), jnp.int32),
            pltpu.VMEM((bpw,), jnp.float32),
            pltpu.VMEM((bpw,), jnp.float32),
            pltpu.VMEM((bpw,), jnp.float32),
            pltpu.SemaphoreType.DMA,
            pltpu.SemaphoreType.DMA,
        ],
    )
    def sc_combine(uid_hbm, iid_hbm, bu_hbm, bi_hbm, fc_hbm, out_hbm,
                   uidx, iidx, urows, irows, fcv, sem_u, sem_i):
        wid = lax.axis_index("s") * nc + lax.axis_index("c")
        base = wid * bpw
        bu_flat = bu_hbm.at[0]
        bi_flat = bi_hbm.at[0]
        pltpu.sync_copy(uid_hbm.at[pl.ds(base, bpw)], uidx)
        pltpu.sync_copy(iid_hbm.at[pl.ds(base, bpw)], iidx)
        cu = pltpu.async_copy(bu_flat.at[uidx], urows, sem_u)
        ci = pltpu.async_copy(bi_flat.at[iidx], irows, sem_i)
        pltpu.sync_copy(fc_hbm.at[pl.ds(base, bpw)], fcv)
        cu.wait()
        ci.wait()
        for j in range(bpw // _LANES):
            sl = pl.ds(j * _LANES, _LANES)
            fcv[sl] = fcv[sl] + urows[sl] + irows[sl]
        pltpu.sync_copy(fcv, out_hbm.at[pl.ds(base, bpw)])

    return sc_combine


_sc_combine = _make_sc_combine()

_TC_BLK = 2048


def _tc_body(w_ref, b_ref, f_ref, o_ref):
    o_ref[...] = jnp.sum(f_ref[...] * w_ref[...], axis=1) + b_ref[0, 0]


def _tc_matvec(fc_weight, fc_bias2, feature):
    return pl.pallas_call(
        _tc_body,
        grid=(B // _TC_BLK,),
        in_specs=[
            pl.BlockSpec((1, D), lambda i: (0, 0)),
            pl.BlockSpec((1, 1), lambda i: (0, 0)),
            pl.BlockSpec((_TC_BLK, D), lambda i: (i, 0)),
        ],
        out_specs=pl.BlockSpec((_TC_BLK,), lambda i: (i,)),
        out_shape=jax.ShapeDtypeStruct((B,), jnp.float32),
    )(fc_weight, fc_bias2, feature)


def kernel(feature, user_id, item_id, fc_weight, fc_bias, b_users, b_items):
    fc1d = _tc_matvec(fc_weight, fc_bias.reshape(1, 1), feature)
    out1d = _sc_combine(user_id, item_id, b_users.T, b_items.T, fc1d)
    return out1d.reshape(B, 1)


# speedup vs baseline: 2.5586x; 1.0569x over previous
import functools

import jax
import jax.numpy as jnp
from jax import lax
from jax.experimental import pallas as pl
from jax.experimental.pallas import tpu as pltpu
from jax.experimental.pallas import tpu_sc as plsc

B = 16384
D = 64
_LANES = 16


def _make_sc_bias():
    info = plsc.get_sparse_core_info()
    nc, ns = info.num_cores, info.num_subcores
    nw = nc * ns
    bpw = B // nw
    mesh = plsc.VectorSubcoreMesh(core_axis_name="c", subcore_axis_name="s")

    @functools.partial(
        pl.kernel,
        mesh=mesh,
        out_type=jax.ShapeDtypeStruct((B,), jnp.float32),
        scratch_types=[
            pltpu.VMEM((bpw,), jnp.int32),
            pltpu.VMEM((bpw,), jnp.int32),
            pltpu.VMEM((bpw,), jnp.float32),
            pltpu.VMEM((bpw,), jnp.float32),
            pltpu.SemaphoreType.DMA,
            pltpu.SemaphoreType.DMA,
        ],
    )
    def sc_bias(uid_hbm, iid_hbm, bu_hbm, bi_hbm, out_hbm,
                uidx, iidx, urows, irows, sem_u, sem_i):
        wid = lax.axis_index("s") * nc + lax.axis_index("c")
        base = wid * bpw
        bu_flat = bu_hbm.at[0]
        bi_flat = bi_hbm.at[0]
        pltpu.sync_copy(uid_hbm.at[pl.ds(base, bpw)], uidx)
        pltpu.sync_copy(iid_hbm.at[pl.ds(base, bpw)], iidx)
        cu = pltpu.async_copy(bu_flat.at[uidx], urows, sem_u)
        ci = pltpu.async_copy(bi_flat.at[iidx], irows, sem_i)
        cu.wait()
        ci.wait()
        for j in range(bpw // _LANES):
            sl = pl.ds(j * _LANES, _LANES)
            urows[sl] = urows[sl] + irows[sl]
        pltpu.sync_copy(urows, out_hbm.at[pl.ds(base, bpw)])

    return sc_bias


_sc_bias = _make_sc_bias()

_TC_BLK = 2048


def _tc_body(w_ref, b_ref, f_ref, o_ref):
    o_ref[...] = jnp.sum(f_ref[...] * w_ref[...], axis=1) + b_ref[0, 0]


def _tc_matvec(fc_weight, fc_bias2, feature):
    return pl.pallas_call(
        _tc_body,
        grid=(B // _TC_BLK,),
        in_specs=[
            pl.BlockSpec((1, D), lambda i: (0, 0)),
            pl.BlockSpec((1, 1), lambda i: (0, 0)),
            pl.BlockSpec((_TC_BLK, D), lambda i: (i, 0)),
        ],
        out_specs=pl.BlockSpec((_TC_BLK,), lambda i: (i,)),
        out_shape=jax.ShapeDtypeStruct((B,), jnp.float32),
    )(fc_weight, fc_bias2, feature)


def _cmb_body(a_ref, b_ref, o_ref):
    o_ref[...] = a_ref[...] + b_ref[...]


def _tc_combine(a, b):
    return pl.pallas_call(
        _cmb_body,
        out_shape=jax.ShapeDtypeStruct((B,), jnp.float32),
    )(a, b)


def kernel(feature, user_id, item_id, fc_weight, fc_bias, b_users, b_items):
    bias1d = _sc_bias(user_id, item_id, b_users.T, b_items.T)
    fc1d = _tc_matvec(fc_weight, fc_bias.reshape(1, 1), feature)
    return _tc_combine(fc1d, bias1d).reshape(B, 1)
